# baseline (speedup 1.0000x reference)
import functools
import math

import jax
import jax.numpy as jnp
from jax import lax
from jax.experimental import pallas as pl
from jax.experimental.pallas import tpu as pltpu
from jax.experimental.pallas import tpu_sc as plsc

D_MODEL = 64
SCALE = math.sqrt(D_MODEL)

NC = 2
NS = 16
NW = NC * NS
LANES = 16
ROW_PAIR = 2 * D_MODEL

CHUNK = 256


def _build(B):
    assert B % NW == 0
    b_per_w = B // NW
    assert b_per_w % CHUNK == 0
    n_chunks = b_per_w // CHUNK

    mesh = plsc.VectorSubcoreMesh(core_axis_name="c", subcore_axis_name="s")

    @functools.partial(
        pl.kernel,
        out_type=jax.ShapeDtypeStruct((B * D_MODEL,), jnp.float32),
        mesh=mesh,
        scratch_types=[
            pltpu.VMEM((b_per_w,), jnp.int32),
            pltpu.VMEM((b_per_w,), jnp.int32),
            pltpu.VMEM((CHUNK, ROW_PAIR), jnp.float32),
            pltpu.VMEM((CHUNK * D_MODEL,), jnp.float32),
            pltpu.SemaphoreType.DMA,
        ],
    )
    def emb(idx_hbm, table2_hbm, out_hbm, idx_v, pair_v, gbuf, wbuf, sem):
        wid = lax.axis_index("s") * NC + lax.axis_index("c")
        base = wid * b_per_w
        pltpu.sync_copy(idx_hbm.at[pl.ds(base, b_per_w)], idx_v)

        def pair_body(i, c):
            sl = pl.ds(i * LANES, LANES)
            pair_v[sl] = idx_v[sl] >> 1
            return c

        lax.fori_loop(0, b_per_w // LANES, pair_body, 0)

        def chunk_body(g, carry):
            pltpu.async_copy(
                table2_hbm.at[pair_v.at[pl.ds(g * CHUNK, CHUNK)]], gbuf, sem
            ).wait()

            def group_body(kk, c):
                off_v = (idx_v[pl.ds(g * CHUNK + kk * LANES, LANES)] & 1) * D_MODEL
                for r in range(LANES):
                    row = kk * LANES + r
                    off = off_v[r]
                    for j in range(D_MODEL // LANES):
                        wbuf[pl.ds(row * D_MODEL + j * LANES, LANES)] = (
                            gbuf[row, pl.ds(off + j * LANES, LANES)] * SCALE
                        )
                return c

            lax.fori_loop(0, CHUNK // LANES, group_body, 0)
            pltpu.sync_copy(
                wbuf,
                out_hbm.at[pl.ds((base + g * CHUNK) * D_MODEL, CHUNK * D_MODEL)],
            )
            return carry

        lax.fori_loop(0, n_chunks, chunk_body, 0)

    return emb


def kernel(x, table):
    orig_shape = x.shape
    xf = x.reshape(-1).astype(jnp.int32)
    table2 = table.reshape(-1, ROW_PAIR)
    out = _build(xf.shape[0])(xf, table2)
    return out.reshape(orig_shape + (D_MODEL,))
